# baseline (speedup 1.0000x reference)
import functools

import jax
import jax.numpy as jnp
from jax import lax
from jax.experimental import pallas as pl
from jax.experimental.pallas import tpu as pltpu
from jax.experimental.pallas import tpu_sc as plsc

N = 10000
E = 320000
NP = 10112
NTILES = 32
CH = 128
NCHUNK = 80
TCHUNK = NTILES * NCHUNK
EP = TCHUNK * CH
CPS = TCHUNK // 2
DH = 48
DO = 8

_mesh = plsc.VectorSubcoreMesh(core_axis_name="c", subcore_axis_name="s")
_sc_params = pltpu.CompilerParams(
    needs_layout_passes=False, use_tc_tiling_on_sc=False)


@functools.partial(
    pl.kernel,
    mesh=_mesh,
    out_type=jax.ShapeDtypeStruct((NTILES * NP,), jnp.float32),
    compiler_params=_sc_params,
    scratch_types=[
        pltpu.VMEM((NCHUNK, CH), jnp.int32),
        pltpu.VMEM((NP,), jnp.float32),
    ],
)
def _deg_kernel(dst_hbm, out_hbm, dst_v, hist_v):
    cid = lax.axis_index("c")
    sid = lax.axis_index("s")
    wid = cid * 16 + sid
    pltpu.sync_copy(dst_hbm.at[pl.ds(wid * NCHUNK, NCHUNK)], dst_v)
    zeros16 = jnp.zeros((16,), jnp.float32)

    def _zero(i, carry):
        hist_v[pl.ds(i * 16, 16)] = zeros16
        return carry

    lax.fori_loop(0, NP // 16, _zero, 0)
    ones16 = jnp.ones((16,), jnp.float32)
    groups = CH // 16

    def _count(i, carry):
        c = i // groups
        k = i % groups
        idx = dst_v[c, pl.ds(k * 16, 16)]
        plsc.addupdate_scatter(hist_v, [idx], ones16)
        return carry

    lax.fori_loop(0, NCHUNK * groups, _count, 0)
    pltpu.sync_copy(hist_v, out_hbm.at[pl.ds(wid * NP, NP)])


def _make_aggv(Wt, G):
    Wg = Wt // G
    H = 16 // G
    CPT = CPS // H
    NBLK = CPT // 16

    @functools.partial(
        pl.kernel,
        mesh=_mesh,
        out_type=jax.ShapeDtypeStruct((2, H, Wt, NP), jnp.float32),
        compiler_params=_sc_params,
        scratch_types=[
            pltpu.VMEM((16, CH), jnp.int32),
            pltpu.VMEM((16, CH), jnp.int32),
            pltpu.VMEM((Wg, NP), jnp.float32),
            pltpu.VMEM((Wg, NP), jnp.float32),
        ],
    )
    def _aggv(tab_hbm, src_hbm, dst_hbm, out_hbm, sblk_v, dblk_v,
              tab_v, acc_v):
        cid = lax.axis_index("c")
        sid = lax.axis_index("s")
        g = sid % G
        h = sid // G
        jvs = [jnp.full((16,), j, jnp.int32) for j in range(Wg)]
        pltpu.sync_copy(tab_hbm.at[pl.ds(g * Wg, Wg)], tab_v)

        zeros16 = jnp.zeros((16,), jnp.float32)
        for r in range(Wg):
            def _zero(i, carry, r=r):
                acc_v[r, pl.ds(i * 16, 16)] = zeros16
                return carry

            lax.fori_loop(0, NP // 16, _zero, 0)

        chunk0 = cid * CPS + h * CPT

        def _block(b, carry):
            base = chunk0 + b * 16
            pltpu.sync_copy(src_hbm.at[pl.ds(base, 16)], sblk_v)
            pltpu.sync_copy(dst_hbm.at[pl.ds(base, 16)], dblk_v)

            def _row(q, c2):
                for k in range(CH // 16):
                    src16 = sblk_v[q, pl.ds(k * 16, 16)]
                    dst16 = dblk_v[q, pl.ds(k * 16, 16)]
                    for j in range(Wg):
                        v = plsc.load_gather(tab_v, [jvs[j], src16])
                        plsc.addupdate_scatter(acc_v, [jvs[j], dst16], v)
                return c2

            lax.fori_loop(0, 16, _row, 0)
            return carry

        lax.fori_loop(0, NBLK, _block, 0)
        pltpu.sync_copy(acc_v, out_hbm.at[cid, h, pl.ds(g * Wg, Wg)])

    return _aggv


_agg_hid = _make_aggv(DH, 8)
_agg_out = _make_aggv(DO, 2)


def _mm1_body(x_ref, w1_ref, degp_ref, hs_ref, dinv_ref):
    deg = jnp.sum(degp_ref[...], axis=0, keepdims=True) + 1.0
    dinv = lax.rsqrt(deg)
    hT = lax.dot_general(w1_ref[...], x_ref[...],
                         (((0,), (1,)), ((), ())),
                         preferred_element_type=jnp.float32)
    hs_ref[...] = hT * dinv
    dinv_ref[...] = dinv


def _mm2_body(s1_ref, hs_ref, dinv_ref, w2_ref, b1_ref, out_ref):
    s1 = s1_ref[0, 0] + s1_ref[0, 1] + s1_ref[1, 0] + s1_ref[1, 1]
    dinv = dinv_ref[...]
    out1 = dinv * (s1 + hs_ref[...]) + b1_ref[...]
    r = jnp.maximum(out1, 0.0)
    h2T = jnp.dot(w2_ref[...], r,
                  preferred_element_type=jnp.float32)
    out_ref[...] = h2T * dinv


def _fin_body(s2_ref, h2s_ref, dinv_ref, b2_ref, out_ref):
    s2 = s2_ref[0, 0]
    for t in range(1, 16):
        s2 = s2 + s2_ref[t // 8, t % 8]
    o = dinv_ref[...] * (s2 + h2s_ref[...]) + b2_ref[...]
    a = o[0:1, :]
    b = o[1:2, :]
    m = jnp.maximum(a, b)
    lse = m + jnp.log(jnp.exp(a - m) + jnp.exp(b - m))
    out_ref[...] = o[0:2, :] - lse


def kernel(x, edge_index, W1, b1, W2, b2):
    src = edge_index[0].astype(jnp.int32)
    dst = edge_index[1].astype(jnp.int32)
    pad = jnp.full((EP - E,), N, jnp.int32)
    srcp = jnp.concatenate([src, pad]).reshape(TCHUNK, CH)
    dstp = jnp.concatenate([dst, pad]).reshape(TCHUNK, CH)
    xp = jnp.pad(x, ((0, NP - N), (0, 0)))
    w1p = jnp.pad(W1, ((0, 0), (0, DH - W1.shape[1])))
    b1c = jnp.pad(b1, (0, DH - b1.shape[0]))[:, None]
    w2t = jnp.pad(W2.T, ((0, DO - W2.shape[1]), (0, DH - W2.shape[0])))
    b2c = jnp.pad(b2, (0, DO - b2.shape[0]))[:, None]

    degp = _deg_kernel(dstp).reshape(NTILES, NP)

    hsT, dinv = pl.pallas_call(
        _mm1_body,
        out_shape=[
            jax.ShapeDtypeStruct((DH, NP), jnp.float32),
            jax.ShapeDtypeStruct((1, NP), jnp.float32),
        ],
    )(xp, w1p, degp)

    s1 = _agg_hid(hsT, srcp, dstp)

    h2sT = pl.pallas_call(
        _mm2_body,
        out_shape=jax.ShapeDtypeStruct((DO, NP), jnp.float32),
    )(s1, hsT, dinv, w2t, b1c)

    s2 = _agg_out(h2sT, srcp, dstp)

    o = pl.pallas_call(
        _fin_body,
        out_shape=jax.ShapeDtypeStruct((2, NP), jnp.float32),
    )(s2, h2sT, dinv, b2c)

    return o.T[:N]


# speedup vs baseline: 1.3155x; 1.3155x over previous
import functools

import jax
import jax.numpy as jnp
from jax import lax
from jax.experimental import pallas as pl
from jax.experimental.pallas import tpu as pltpu
from jax.experimental.pallas import tpu_sc as plsc

N = 10000
E = 320000
NP = 10112
NTILES = 32
CH = 128
NCHUNK = 80
K = 8
NC0 = 64
NC1 = 96
TCHUNK = NTILES * NCHUNK
EP = NTILES * NCHUNK * CH
DH = 48
DO = 8
RPT = NP // 16

_mesh = plsc.VectorSubcoreMesh(core_axis_name="c", subcore_axis_name="s")
_sc_params = pltpu.CompilerParams(
    needs_layout_passes=False, use_tc_tiling_on_sc=False)


@functools.partial(
    pl.kernel,
    mesh=_mesh,
    out_type=jax.ShapeDtypeStruct((NTILES * NP,), jnp.float32),
    compiler_params=_sc_params,
    scratch_types=[
        pltpu.VMEM((NCHUNK, CH), jnp.int32),
        pltpu.VMEM((NP,), jnp.float32),
    ],
)
def _deg_kernel(dst_hbm, out_hbm, dst_v, hist_v):
    cid = lax.axis_index("c")
    sid = lax.axis_index("s")
    wid = cid * 16 + sid
    pltpu.sync_copy(dst_hbm.at[pl.ds(wid * NCHUNK, NCHUNK)], dst_v)
    zeros16 = jnp.zeros((16,), jnp.float32)

    def _zero(i, carry):
        hist_v[pl.ds(i * 16, 16)] = zeros16
        return carry

    lax.fori_loop(0, NP // 16, _zero, 0)
    ones16 = jnp.ones((16,), jnp.float32)
    groups = CH // 16

    def _count(i, carry):
        c = i // groups
        k = i % groups
        idx = dst_v[c, pl.ds(k * 16, 16)]
        plsc.addupdate_scatter(hist_v, [idx], ones16)
        return carry

    lax.fori_loop(0, NCHUNK * groups, _count, 0)
    pltpu.sync_copy(hist_v, out_hbm.at[pl.ds(wid * NP, NP)])


def _make_agg(W):
    @functools.partial(
        pl.kernel,
        mesh=_mesh,
        out_type=jax.ShapeDtypeStruct((2, NP, W), jnp.float32),
        compiler_params=_sc_params,
        scratch_types=[
            pltpu.VMEM((NC1, CH), jnp.int32),
            pltpu.VMEM((NC1, CH), jnp.int32),
            pltpu.VMEM((K, CH, W), jnp.float32),
            pltpu.VMEM((CH, W), jnp.float32),
            pltpu.VMEM_SHARED((NP, W), jnp.float32),
            pltpu.SemaphoreType.DMA,
            pltpu.SemaphoreType.DMA,
        ],
    )
    def _agg(tab_hbm, src_hbm, dst_hbm, zeros_hbm, out_hbm,
             src_v, dst_v, rows_v, zero_v, acc_sh, gsem, ssem):
        cid = lax.axis_index("c")
        sid = lax.axis_index("s")
        start = jnp.where(cid == 0, sid * NC0, 16 * NC0 + sid * NC1)
        nbatch = jnp.where(cid == 0, NC0 // K, NC1 // K)

        @pl.when(cid == 0)
        def _():
            pltpu.sync_copy(src_hbm.at[pl.ds(start, NC0)],
                            src_v.at[pl.ds(0, NC0)])
            pltpu.sync_copy(dst_hbm.at[pl.ds(start, NC0)],
                            dst_v.at[pl.ds(0, NC0)])

        @pl.when(cid == 1)
        def _():
            pltpu.sync_copy(src_hbm.at[pl.ds(start, NC1)], src_v)
            pltpu.sync_copy(dst_hbm.at[pl.ds(start, NC1)], dst_v)

        pltpu.sync_copy(zeros_hbm, zero_v)

        base = sid * RPT
        off = 0
        while off < RPT:
            n = min(CH, RPT - off)
            pltpu.sync_copy(zero_v.at[pl.ds(0, n)],
                            acc_sh.at[pl.ds(base + off, n)])
            off += n
        plsc.subcore_barrier()

        def _batch(g, carry):
            c0 = g * K
            for j in range(K):
                pltpu.async_copy(
                    tab_hbm.at[src_v.at[c0 + j]], rows_v.at[j], gsem)
            for j in range(K):
                pltpu.make_async_copy(
                    tab_hbm.at[src_v.at[c0 + j]], rows_v.at[j], gsem).wait()
            for j in range(K):
                pltpu.async_copy(
                    rows_v.at[j], acc_sh.at[dst_v.at[c0 + j]], ssem, add=True)
            for j in range(K):
                pltpu.make_async_copy(
                    rows_v.at[j], acc_sh.at[dst_v.at[c0 + j]], ssem).wait()
            return carry

        lax.fori_loop(0, nbatch, _batch, 0)
        plsc.subcore_barrier()
        pltpu.sync_copy(acc_sh.at[pl.ds(base, RPT)],
                        out_hbm.at[cid, pl.ds(base, RPT)])

    return _agg


_agg_hid = _make_agg(DH)
_agg_out = _make_agg(DO)


def _mm1_body(x_ref, w1_ref, degp_ref, hs_ref, dinv_ref):
    deg = jnp.sum(degp_ref[...], axis=1, keepdims=True) + 1.0
    dinv = lax.rsqrt(deg)
    h = jnp.dot(x_ref[...], w1_ref[...], preferred_element_type=jnp.float32)
    hs_ref[pl.ds(0, N), :] = h * dinv[:N]
    hs_ref[pl.ds(N, NP - N), :] = jnp.zeros((NP - N, DH), jnp.float32)
    dinv_ref[...] = dinv


def _mm2_body(s1_ref, hs_ref, dinv_ref, w2_ref, b1_ref, out_ref):
    s1 = s1_ref[0] + s1_ref[1]
    dinv = dinv_ref[...]
    out1 = dinv * (s1 + hs_ref[...]) + b1_ref[...]
    r = jnp.maximum(out1, 0.0)
    h2 = jnp.dot(r, w2_ref[...], preferred_element_type=jnp.float32)
    out_ref[...] = h2 * dinv


def _fin_body(s2_ref, h2s_ref, dinv_ref, b2_ref, out_ref):
    s2 = s2_ref[0] + s2_ref[1]
    o = dinv_ref[...] * (s2 + h2s_ref[...]) + b2_ref[...]
    a = o[:, 0:1]
    b = o[:, 1:2]
    m = jnp.maximum(a, b)
    lse = m + jnp.log(jnp.exp(a - m) + jnp.exp(b - m))
    out_ref[...] = o - lse


def kernel(x, edge_index, W1, b1, W2, b2):
    src = edge_index[0].astype(jnp.int32)
    dst = edge_index[1].astype(jnp.int32)
    pad = jnp.full((EP - E,), N, jnp.int32)
    srcp = jnp.concatenate([src, pad]).reshape(TCHUNK, CH)
    dstp = jnp.concatenate([dst, pad]).reshape(TCHUNK, CH)
    w1p = jnp.pad(W1, ((0, 0), (0, DH - W1.shape[1])))
    b1p = jnp.pad(b1, (0, DH - b1.shape[0]))[None, :]
    w2p = jnp.pad(W2, ((0, DH - W2.shape[0]), (0, DO - W2.shape[1])))
    b2p = jnp.pad(b2, (0, DO - b2.shape[0]))[None, :]
    z1 = jnp.zeros((CH, DH), jnp.float32)
    z2 = jnp.zeros((CH, DO), jnp.float32)

    degp = _deg_kernel(dstp)
    degp_t = degp.reshape(NTILES, NP).T

    hs, dinv = pl.pallas_call(
        _mm1_body,
        out_shape=[
            jax.ShapeDtypeStruct((NP, DH), jnp.float32),
            jax.ShapeDtypeStruct((NP, 1), jnp.float32),
        ],
    )(x, w1p, degp_t)

    s1 = _agg_hid(hs, srcp, dstp, z1)

    h2s = pl.pallas_call(
        _mm2_body,
        out_shape=jax.ShapeDtypeStruct((NP, DO), jnp.float32),
    )(s1, hs, dinv, w2p, b1p)

    s2 = _agg_out(h2s, srcp, dstp, z2)

    o = pl.pallas_call(
        _fin_body,
        out_shape=jax.ShapeDtypeStruct((NP, DO), jnp.float32),
    )(s2, h2s, dinv, b2p)

    return o[:N, :2]
